# baseline (speedup 1.0000x reference)
import jax
import jax.numpy as jnp
from jax import lax
from jax.experimental import pallas as pl
from jax.experimental.pallas import tpu as pltpu
from jax.experimental.pallas import tpu_sc as plsc

NC = 2
NS = 16
L = 16
NW = NC * NS

T = 2097152
K = 1048576
D = 3

KW = K // NW
C = 8192
NCHUNK = KW // C


def _sc_body(flat_hbm, idx_hbm, dx_hbm, dy_hbm, out_hbm,
             idx_v, ix_v, iy_v, xs_v, ys_v, dxv, dyv, accv, sem):
    wid = lax.axis_index("s") * NC + lax.axis_index("c")
    base = wid * KW

    pltpu.sync_copy(dx_hbm, dxv)
    pltpu.sync_copy(dy_hbm, dyv)
    dx_t = dxv[...]
    dy_t = dyv[...]

    three = jnp.full((L,), 3, jnp.int32)
    one = jnp.full((L,), 1, jnp.int32)

    def chunk_body(g, acc):
        pltpu.sync_copy(idx_hbm.at[pl.ds(base + g * C, C)], idx_v)

        def trip(i, carry):
            i3 = idx_v[pl.ds(i * L, L)] * three
            ix_v[pl.ds(i * L, L)] = i3
            iy_v[pl.ds(i * L, L)] = i3 + one
            return carry

        lax.fori_loop(0, C // L, trip, 0)

        cx = pltpu.async_copy(flat_hbm.at[ix_v], xs_v, sem)
        cy = pltpu.async_copy(flat_hbm.at[iy_v], ys_v, sem)
        cx.wait()
        cy.wait()

        def inner(i, a):
            x = xs_v[pl.ds(i * L, L)]
            y = ys_v[pl.ds(i * L, L)]
            ddx = x - dx_t
            ddy = y - dy_t
            return a + ddx * ddx + ddy * ddy

        return lax.fori_loop(0, C // L, inner, acc)

    acc = lax.fori_loop(0, NCHUNK, chunk_body, jnp.zeros((L,), jnp.float32))
    accv[...] = acc
    pltpu.sync_copy(accv, out_hbm.at[wid])


@jax.jit
def _sc_partials(flat, idx, dxv, dyv):
    mesh = plsc.VectorSubcoreMesh(core_axis_name="c", subcore_axis_name="s")
    return pl.kernel(
        _sc_body,
        out_type=jax.ShapeDtypeStruct((NW, L), jnp.float32),
        mesh=mesh,
        scratch_types=[
            pltpu.VMEM((C,), jnp.int32),
            pltpu.VMEM((C,), jnp.int32),
            pltpu.VMEM((C,), jnp.int32),
            pltpu.VMEM((C,), jnp.float32),
            pltpu.VMEM((C,), jnp.float32),
            pltpu.VMEM((L,), jnp.float32),
            pltpu.VMEM((L,), jnp.float32),
            pltpu.VMEM((L,), jnp.float32),
            pltpu.SemaphoreType.DMA,
        ],
        compiler_params=pltpu.CompilerParams(
            needs_layout_passes=False, use_tc_tiling_on_sc=False),
    )(flat, idx, dxv, dyv)


def kernel(xt, indices, dest):
    idx = indices.astype(jnp.int32)
    flat = xt.reshape(T * D)
    dxv = jnp.full((L,), dest[0, 0], dtype=jnp.float32)
    dyv = jnp.full((L,), dest[0, 1], dtype=jnp.float32)
    partials = _sc_partials(flat, idx, dxv, dyv)
    return jnp.sum(partials)


# speedup vs baseline: 22.3386x; 22.3386x over previous
import jax
import jax.numpy as jnp
from jax import lax
from jax.experimental import pallas as pl
from jax.experimental.pallas import tpu as pltpu
from jax.experimental.pallas import tpu_sc as plsc

NC = 2
NS = 16
L = 16
NW = NC * NS

T = 2097152
K = 1048576
D = 3

KW = K // NW
U = 8


def _sc_body(x_hbm, y_hbm, idx_hbm, dx_hbm, dy_hbm, out_hbm,
             idx_v, xs_v, ys_v, dxv, dyv, accv, sem):
    wid = lax.axis_index("s") * NC + lax.axis_index("c")
    base = wid * KW

    pltpu.sync_copy(idx_hbm.at[pl.ds(base, KW)], idx_v)
    cx = pltpu.async_copy(x_hbm.at[idx_v], xs_v, sem)
    cy = pltpu.async_copy(y_hbm.at[idx_v], ys_v, sem)

    pltpu.sync_copy(dx_hbm, dxv)
    pltpu.sync_copy(dy_hbm, dyv)
    dx_t = dxv[...]
    dy_t = dyv[...]

    cx.wait()
    cy.wait()

    zero = jnp.zeros((L,), jnp.float32)

    def inner(i, accs):
        b = i * (U * L)
        out = []
        for u in range(U):
            x = xs_v[pl.ds(b + u * L, L)]
            y = ys_v[pl.ds(b + u * L, L)]
            ddx = x - dx_t
            ddy = y - dy_t
            out.append(accs[u] + ddx * ddx + ddy * ddy)
        return tuple(out)

    accs = lax.fori_loop(0, KW // (U * L), inner, (zero,) * U)
    acc = accs[0]
    for u in range(1, U):
        acc = acc + accs[u]
    accv[...] = acc
    pltpu.sync_copy(accv, out_hbm.at[wid])


@jax.jit
def _sc_partials(xcol, ycol, idx, dxv, dyv):
    mesh = plsc.VectorSubcoreMesh(core_axis_name="c", subcore_axis_name="s")
    return pl.kernel(
        _sc_body,
        out_type=jax.ShapeDtypeStruct((NW, L), jnp.float32),
        mesh=mesh,
        scratch_types=[
            pltpu.VMEM((KW,), jnp.int32),
            pltpu.VMEM((KW,), jnp.float32),
            pltpu.VMEM((KW,), jnp.float32),
            pltpu.VMEM((L,), jnp.float32),
            pltpu.VMEM((L,), jnp.float32),
            pltpu.VMEM((L,), jnp.float32),
            pltpu.SemaphoreType.DMA,
        ],
        compiler_params=pltpu.CompilerParams(
            needs_layout_passes=False, use_tc_tiling_on_sc=False),
    )(xcol, ycol, idx, dxv, dyv)


def kernel(xt, indices, dest):
    idx = indices.astype(jnp.int32)
    xcol = xt[:, 0]
    ycol = xt[:, 1]
    dxv = jnp.full((L,), dest[0, 0], dtype=jnp.float32)
    dyv = jnp.full((L,), dest[0, 1], dtype=jnp.float32)
    partials = _sc_partials(xcol, ycol, idx, dxv, dyv)
    return jnp.sum(partials)


# speedup vs baseline: 22.3715x; 1.0015x over previous
import jax
import jax.numpy as jnp
from jax import lax
from jax.experimental import pallas as pl
from jax.experimental.pallas import tpu as pltpu
from jax.experimental.pallas import tpu_sc as plsc

NC = 2
NS = 16
L = 16
NW = NC * NS

T = 2097152
K = 1048576
D = 3

KW = K // NW
U = 8


NH = 2
H = KW // NH


def _sc_body(x_hbm, y_hbm, idx_hbm, dx_hbm, dy_hbm, out_hbm,
             idx_v, xs_v, ys_v, dxv, dyv, accv,
             sem0, sem1, sem2, sem3):
    wid = lax.axis_index("s") * NC + lax.axis_index("c")
    base = wid * KW
    sems = (sem0, sem1, sem2, sem3)

    pltpu.sync_copy(idx_hbm.at[pl.ds(base, KW)], idx_v)
    copies = []
    for h in range(NH):
        sl = pl.ds(h * H, H)
        copies.append(pltpu.async_copy(
            x_hbm.at[idx_v.at[sl]], xs_v.at[sl], sems[2 * h]))
        copies.append(pltpu.async_copy(
            y_hbm.at[idx_v.at[sl]], ys_v.at[sl], sems[2 * h + 1]))

    pltpu.sync_copy(dx_hbm, dxv)
    pltpu.sync_copy(dy_hbm, dyv)
    dx_t = dxv[...]
    dy_t = dyv[...]

    zero = jnp.zeros((L,), jnp.float32)

    def make_pass(src_v, tgt):
        def body(i, accs):
            b = i * (U * L)
            out = []
            for u in range(U):
                d = src_v[pl.ds(b + u * L, L)] - tgt
                out.append(accs[u] + d * d)
            return tuple(out)
        return body

    accs = (zero,) * U
    for h in range(NH):
        lo = h * H // (U * L)
        hi = (h + 1) * H // (U * L)
        copies[2 * h].wait()
        accs = lax.fori_loop(lo, hi, make_pass(xs_v, dx_t), accs)
        copies[2 * h + 1].wait()
        accs = lax.fori_loop(lo, hi, make_pass(ys_v, dy_t), accs)

    acc = accs[0]
    for u in range(1, U):
        acc = acc + accs[u]
    accv[...] = acc
    pltpu.sync_copy(accv, out_hbm.at[wid])


@jax.jit
def _sc_partials(xcol, ycol, idx, dxv, dyv):
    mesh = plsc.VectorSubcoreMesh(core_axis_name="c", subcore_axis_name="s")
    return pl.kernel(
        _sc_body,
        out_type=jax.ShapeDtypeStruct((NW, L), jnp.float32),
        mesh=mesh,
        scratch_types=[
            pltpu.VMEM((KW,), jnp.int32),
            pltpu.VMEM((KW,), jnp.float32),
            pltpu.VMEM((KW,), jnp.float32),
            pltpu.VMEM((L,), jnp.float32),
            pltpu.VMEM((L,), jnp.float32),
            pltpu.VMEM((L,), jnp.float32),
            pltpu.SemaphoreType.DMA,
            pltpu.SemaphoreType.DMA,
            pltpu.SemaphoreType.DMA,
            pltpu.SemaphoreType.DMA,
        ],
        compiler_params=pltpu.CompilerParams(
            needs_layout_passes=False, use_tc_tiling_on_sc=False),
    )(xcol, ycol, idx, dxv, dyv)


def kernel(xt, indices, dest):
    idx = indices.astype(jnp.int32)
    xcol = xt[:, 0]
    ycol = xt[:, 1]
    dxv = jnp.full((L,), dest[0, 0], dtype=jnp.float32)
    dyv = jnp.full((L,), dest[0, 1], dtype=jnp.float32)
    partials = _sc_partials(xcol, ycol, idx, dxv, dyv)
    return jnp.sum(partials)
